# baseline (speedup 1.0000x reference)
import functools

import jax
import jax.numpy as jnp
from jax import lax
from jax.experimental import pallas as pl
from jax.experimental.pallas import tpu as pltpu
from jax.experimental.pallas import tpu_sc as plsc

N = 10000
D = 128
E = 320000
NC = 2
NS = 16
NW = NC * NS
EPW = E // NW
K = 40
NCHUNK = EPW // K
NSUP = 5
CPS = NCHUNK // NSUP
NBUF = 4
PF = NBUF - 1
KD = 80
CPSD = EPW // KD // NSUP
RPT = N // NS
ZROWS = 125

_mesh = plsc.VectorSubcoreMesh(core_axis_name="c", subcore_axis_name="s")


@functools.partial(
    pl.kernel,
    out_type=jax.ShapeDtypeStruct((NC, 1, N), jnp.float32),
    mesh=_mesh,
    scratch_types=[
        pltpu.VMEM((CPSD, KD), jnp.int32),
        pltpu.VMEM((KD,), jnp.float32),
        pltpu.VMEM((N,), jnp.float32),
        pltpu.VMEM_SHARED((N,), jnp.float32),
    ],
)
def _deg_kernel(dst_hbm, out_hbm, didx, ones_v, zbuf, acc):
    c = lax.axis_index("c")
    s = lax.axis_index("s")
    wid = c * NS + s
    for i in range(KD // 16):
        ones_v[pl.ds(i * 16, 16)] = jnp.ones((16,), jnp.float32)

    @pl.when(s == 0)
    def _():
        def zr(i, carry):
            zbuf[pl.ds(i * 16, 16)] = jnp.zeros((16,), jnp.float32)
            return carry
        lax.fori_loop(0, N // 16, zr, 0)
        pltpu.sync_copy(zbuf, acc)

    plsc.subcore_barrier()

    def sup(sc, carry):
        pltpu.sync_copy(dst_hbm.at[wid, sc], didx)

        def body(j, carry2):
            pltpu.sync_copy(ones_v, acc.at[didx.at[j]], add=True)
            return carry2
        lax.fori_loop(0, CPSD, body, 0)
        return carry
    lax.fori_loop(0, NSUP, sup, 0)

    plsc.subcore_barrier()

    @pl.when(s == 0)
    def _():
        pltpu.sync_copy(acc, out_hbm.at[c, 0])


@functools.partial(
    pl.kernel,
    out_type=jax.ShapeDtypeStruct((NC, NS, RPT, D), jnp.float32),
    mesh=_mesh,
    scratch_types=[
        pltpu.VMEM((CPS, K), jnp.int32),
        pltpu.VMEM((CPS, K), jnp.int32),
        pltpu.VMEM((NBUF, K, D), jnp.float32),
        pltpu.VMEM_SHARED((N, D), jnp.float32),
        pltpu.SemaphoreType.DMA((NBUF,)),
        pltpu.SemaphoreType.DMA((NBUF,)),
    ],
)
def _edge_kernel(g_hbm, src_hbm, dst_hbm, out_hbm, sidx, didx, rows, acc, gsem,
                 ssem):
    c = lax.axis_index("c")
    s = lax.axis_index("s")
    wid = c * NS + s

    def zr(i, carry):
        for jj in range(D // 16):
            rows[0, i, pl.ds(jj * 16, 16)] = jnp.zeros((16,), jnp.float32)
        return carry
    lax.fori_loop(0, K, zr, 0)
    for t in range(RPT // K):
        pltpu.sync_copy(rows.at[0], acc.at[pl.ds(s * RPT + t * K, K)])
    pltpu.sync_copy(rows.at[0, pl.ds(0, RPT % K)],
                    acc.at[pl.ds(s * RPT + (RPT // K) * K, RPT % K)])

    plsc.subcore_barrier()

    def sup(sc, carry):
        pltpu.sync_copy(src_hbm.at[wid, sc], sidx)
        pltpu.sync_copy(dst_hbm.at[wid, sc], didx)

        for i in range(PF):
            pltpu.async_copy(g_hbm.at[sidx.at[i]], rows.at[i], gsem.at[i])

        def body(j, carry2):
            rb = lax.rem(j, NBUF)
            nb = lax.rem(j + PF, NBUF)
            pltpu.make_async_copy(g_hbm.at[sidx.at[j]], rows.at[rb],
                                  gsem.at[rb]).wait()

            @pl.when(jnp.logical_and(j >= 1, j + PF < CPS))
            def _():
                pltpu.make_async_copy(rows.at[nb], acc.at[didx.at[j - 1]],
                                      ssem.at[nb]).wait()

            @pl.when(j + PF < CPS)
            def _():
                pltpu.async_copy(g_hbm.at[sidx.at[j + PF]], rows.at[nb],
                                 gsem.at[nb])

            pltpu.async_copy(rows.at[rb], acc.at[didx.at[j]], ssem.at[rb],
                             add=True)
            return carry2
        lax.fori_loop(0, CPS, body, 0)

        for jj in range(CPS - NBUF, CPS):
            pltpu.make_async_copy(rows.at[jj % NBUF], acc.at[didx.at[jj]],
                                  ssem.at[jj % NBUF]).wait()
        return carry
    lax.fori_loop(0, NSUP, sup, 0)

    plsc.subcore_barrier()
    pltpu.sync_copy(acc.at[pl.ds(s * RPT, RPT)], out_hbm.at[c, s])


BM = 1000


def _mm1_body(x_ref, w_ref, degt_ref, g_ref):
    deg = degt_ref[:, 0] + degt_ref[:, 1] + 1.0
    dis = lax.rsqrt(deg)
    h = jnp.dot(x_ref[...], w_ref[...], preferred_element_type=jnp.float32)
    g_ref[...] = h * dis[:, None]


def _mm2_body(p_ref, g1_ref, degt_ref, b1_ref, w_ref, g_ref):
    deg = degt_ref[:, 0] + degt_ref[:, 1] + 1.0
    dis = lax.rsqrt(deg)
    u = jnp.maximum(
        dis[:, None] * (p_ref[0] + p_ref[1] + g1_ref[...]) + b1_ref[...], 0.0)
    h = jnp.dot(u, w_ref[...], preferred_element_type=jnp.float32)
    g_ref[...] = h * dis[:, None]


def _fin_body(q_ref, g2_ref, degt_ref, b2_ref, z_ref):
    deg = degt_ref[:, 0] + degt_ref[:, 1] + 1.0
    dis = lax.rsqrt(deg)
    z_ref[...] = dis[:, None] * (q_ref[0] + q_ref[1] + g2_ref[...]) + b2_ref[...]


def _mm1(x, W1, degt):
    return pl.pallas_call(
        _mm1_body,
        grid=(N // BM,),
        in_specs=[
            pl.BlockSpec((BM, D), lambda i: (i, 0)),
            pl.BlockSpec((D, D), lambda i: (0, 0)),
            pl.BlockSpec((BM, 2), lambda i: (i, 0)),
        ],
        out_specs=pl.BlockSpec((BM, D), lambda i: (i, 0)),
        out_shape=jax.ShapeDtypeStruct((N, D), jnp.float32),
    )(x, W1, degt)


def _mm2(p, g1, degt, b1, W2):
    return pl.pallas_call(
        _mm2_body,
        grid=(N // BM,),
        in_specs=[
            pl.BlockSpec((NC, BM, D), lambda i: (0, i, 0)),
            pl.BlockSpec((BM, D), lambda i: (i, 0)),
            pl.BlockSpec((BM, 2), lambda i: (i, 0)),
            pl.BlockSpec((1, D), lambda i: (0, 0)),
            pl.BlockSpec((D, D), lambda i: (0, 0)),
        ],
        out_specs=pl.BlockSpec((BM, D), lambda i: (i, 0)),
        out_shape=jax.ShapeDtypeStruct((N, D), jnp.float32),
    )(p, g1, degt, b1, W2)


def _fin(q, g2, degt, b2):
    return pl.pallas_call(
        _fin_body,
        grid=(N // BM,),
        in_specs=[
            pl.BlockSpec((NC, BM, D), lambda i: (0, i, 0)),
            pl.BlockSpec((BM, D), lambda i: (i, 0)),
            pl.BlockSpec((BM, 2), lambda i: (i, 0)),
            pl.BlockSpec((1, D), lambda i: (0, 0)),
        ],
        out_specs=pl.BlockSpec((BM, D), lambda i: (i, 0)),
        out_shape=jax.ShapeDtypeStruct((N, D), jnp.float32),
    )(q, g2, degt, b2)


def kernel(x, edge_index, W1, b1, W2, b2):
    ei = edge_index.astype(jnp.int32)
    src = ei[0].reshape(NW, NSUP, CPS, K)
    dst = ei[1].reshape(NW, NSUP, CPS, K)

    dstd = ei[1].reshape(NW, NSUP, CPSD, KD)
    degp = _deg_kernel(dstd)
    degt = degp.reshape(NC, N).T
    g1 = _mm1(x, W1, degt)
    p = _edge_kernel(g1, src, dst).reshape(NC, N, D)
    g2 = _mm2(p, g1, degt, b1.reshape(1, D), W2)
    q = _edge_kernel(g2, src, dst).reshape(NC, N, D)
    return _fin(q, g2, degt, b2.reshape(1, D))


# speedup vs baseline: 1.1046x; 1.1046x over previous
import functools

import jax
import jax.numpy as jnp
from jax import lax
from jax.experimental import pallas as pl
from jax.experimental.pallas import tpu as pltpu
from jax.experimental.pallas import tpu_sc as plsc

N = 10000
D = 128
E = 320000
NC = 2
NS = 16
NW = NC * NS
EPW = E // NW
K = 80
NCHUNK = EPW // K
NSUP = 5
CPS = NCHUNK // NSUP
NBUF = 3
PF = NBUF - 1
KD = 80
CPSD = EPW // KD // NSUP
RPT = N // NS
ZROWS = 125

_mesh = plsc.VectorSubcoreMesh(core_axis_name="c", subcore_axis_name="s")


@functools.partial(
    pl.kernel,
    out_type=jax.ShapeDtypeStruct((NC, 1, N), jnp.float32),
    mesh=_mesh,
    scratch_types=[
        pltpu.VMEM((CPSD, KD), jnp.int32),
        pltpu.VMEM((KD,), jnp.float32),
        pltpu.VMEM((N,), jnp.float32),
        pltpu.VMEM_SHARED((N,), jnp.float32),
    ],
)
def _deg_kernel(dst_hbm, out_hbm, didx, ones_v, zbuf, acc):
    c = lax.axis_index("c")
    s = lax.axis_index("s")
    wid = c * NS + s
    for i in range(KD // 16):
        ones_v[pl.ds(i * 16, 16)] = jnp.ones((16,), jnp.float32)

    @pl.when(s == 0)
    def _():
        def zr(i, carry):
            zbuf[pl.ds(i * 16, 16)] = jnp.zeros((16,), jnp.float32)
            return carry
        lax.fori_loop(0, N // 16, zr, 0)
        pltpu.sync_copy(zbuf, acc)

    plsc.subcore_barrier()

    def sup(sc, carry):
        pltpu.sync_copy(dst_hbm.at[wid, sc], didx)

        def body(j, carry2):
            pltpu.sync_copy(ones_v, acc.at[didx.at[j]], add=True)
            return carry2
        lax.fori_loop(0, CPSD, body, 0)
        return carry
    lax.fori_loop(0, NSUP, sup, 0)

    plsc.subcore_barrier()

    @pl.when(s == 0)
    def _():
        pltpu.sync_copy(acc, out_hbm.at[c, 0])


@functools.partial(
    pl.kernel,
    out_type=jax.ShapeDtypeStruct((NC, N, D), jnp.float32),
    mesh=_mesh,
    scratch_types=[
        pltpu.VMEM((CPS, K), jnp.int32),
        pltpu.VMEM((CPS, K), jnp.int32),
        pltpu.VMEM((NBUF, K, D), jnp.float32),
        pltpu.VMEM_SHARED((N, D), jnp.float32),
        pltpu.SemaphoreType.DMA((NBUF,)),
        pltpu.SemaphoreType.DMA((NBUF,)),
    ],
)
def _edge_kernel(g_hbm, src_hbm, dst_hbm, out_hbm, sidx, didx, rows, acc, gsem,
                 ssem):
    c = lax.axis_index("c")
    s = lax.axis_index("s")
    wid = c * NS + s

    def zr(i, carry):
        for jj in range(D // 16):
            rows[0, i, pl.ds(jj * 16, 16)] = jnp.zeros((16,), jnp.float32)
        return carry
    lax.fori_loop(0, K, zr, 0)
    for t in range(RPT // K):
        pltpu.sync_copy(rows.at[0], acc.at[pl.ds(s * RPT + t * K, K)])
    pltpu.sync_copy(rows.at[0, pl.ds(0, RPT % K)],
                    acc.at[pl.ds(s * RPT + (RPT // K) * K, RPT % K)])

    plsc.subcore_barrier()

    def sup(sc, carry):
        pltpu.sync_copy(src_hbm.at[wid, sc], sidx)
        pltpu.sync_copy(dst_hbm.at[wid, sc], didx)

        for i in range(PF):
            pltpu.async_copy(g_hbm.at[sidx.at[i]], rows.at[i], gsem.at[i])

        def body(j, carry2):
            rb = lax.rem(j, NBUF)
            nb = lax.rem(j + PF, NBUF)
            pltpu.make_async_copy(g_hbm.at[sidx.at[j]], rows.at[rb],
                                  gsem.at[rb]).wait()

            @pl.when(jnp.logical_and(j >= 1, j + PF < CPS))
            def _():
                pltpu.make_async_copy(rows.at[nb], acc.at[didx.at[j - 1]],
                                      ssem.at[nb]).wait()

            @pl.when(j + PF < CPS)
            def _():
                pltpu.async_copy(g_hbm.at[sidx.at[j + PF]], rows.at[nb],
                                 gsem.at[nb])

            pltpu.async_copy(rows.at[rb], acc.at[didx.at[j]], ssem.at[rb],
                             add=True)
            return carry2
        lax.fori_loop(0, CPS, body, 0)

        for jj in range(CPS - NBUF, CPS):
            pltpu.make_async_copy(rows.at[jj % NBUF], acc.at[didx.at[jj]],
                                  ssem.at[jj % NBUF]).wait()
        return carry
    lax.fori_loop(0, NSUP, sup, 0)

    plsc.subcore_barrier()
    WA = (N // NS) // 8 * 8

    @pl.when(s < NS - 1)
    def _():
        pltpu.sync_copy(acc.at[pl.ds(s * WA, WA)], out_hbm.at[c, pl.ds(s * WA, WA)])

    @pl.when(s == NS - 1)
    def _():
        pltpu.sync_copy(acc.at[pl.ds((NS - 1) * WA, N - (NS - 1) * WA)],
                        out_hbm.at[c, pl.ds((NS - 1) * WA, N - (NS - 1) * WA)])


BM = 2000


def _mm1_body(x_ref, w_ref, degt_ref, g_ref):
    deg = degt_ref[:, 0] + degt_ref[:, 1] + 1.0
    dis = lax.rsqrt(deg)
    h = jnp.dot(x_ref[...], w_ref[...], preferred_element_type=jnp.float32)
    g_ref[...] = h * dis[:, None]


def _mm2_body(p_ref, g1_ref, degt_ref, b1_ref, w_ref, g_ref):
    deg = degt_ref[:, 0] + degt_ref[:, 1] + 1.0
    dis = lax.rsqrt(deg)
    u = jnp.maximum(
        dis[:, None] * (p_ref[0] + p_ref[1] + g1_ref[...]) + b1_ref[...], 0.0)
    h = jnp.dot(u, w_ref[...], preferred_element_type=jnp.float32)
    g_ref[...] = h * dis[:, None]


def _fin_body(q_ref, g2_ref, degt_ref, b2_ref, z_ref):
    deg = degt_ref[:, 0] + degt_ref[:, 1] + 1.0
    dis = lax.rsqrt(deg)
    z_ref[...] = dis[:, None] * (q_ref[0] + q_ref[1] + g2_ref[...]) + b2_ref[...]


def _mm1(x, W1, degt):
    return pl.pallas_call(
        _mm1_body,
        grid=(N // BM,),
        in_specs=[
            pl.BlockSpec((BM, D), lambda i: (i, 0)),
            pl.BlockSpec((D, D), lambda i: (0, 0)),
            pl.BlockSpec((BM, 2), lambda i: (i, 0)),
        ],
        out_specs=pl.BlockSpec((BM, D), lambda i: (i, 0)),
        out_shape=jax.ShapeDtypeStruct((N, D), jnp.float32),
    )(x, W1, degt)


def _mm2(p, g1, degt, b1, W2):
    return pl.pallas_call(
        _mm2_body,
        grid=(N // BM,),
        in_specs=[
            pl.BlockSpec((NC, BM, D), lambda i: (0, i, 0)),
            pl.BlockSpec((BM, D), lambda i: (i, 0)),
            pl.BlockSpec((BM, 2), lambda i: (i, 0)),
            pl.BlockSpec((1, D), lambda i: (0, 0)),
            pl.BlockSpec((D, D), lambda i: (0, 0)),
        ],
        out_specs=pl.BlockSpec((BM, D), lambda i: (i, 0)),
        out_shape=jax.ShapeDtypeStruct((N, D), jnp.float32),
    )(p, g1, degt, b1, W2)


def _fin(q, g2, degt, b2):
    return pl.pallas_call(
        _fin_body,
        grid=(N // BM,),
        in_specs=[
            pl.BlockSpec((NC, BM, D), lambda i: (0, i, 0)),
            pl.BlockSpec((BM, D), lambda i: (i, 0)),
            pl.BlockSpec((BM, 2), lambda i: (i, 0)),
            pl.BlockSpec((1, D), lambda i: (0, 0)),
        ],
        out_specs=pl.BlockSpec((BM, D), lambda i: (i, 0)),
        out_shape=jax.ShapeDtypeStruct((N, D), jnp.float32),
    )(q, g2, degt, b2)


def kernel(x, edge_index, W1, b1, W2, b2):
    ei = edge_index.astype(jnp.int32)
    src = ei[0].reshape(NW, NSUP, CPS, K)
    dst = ei[1].reshape(NW, NSUP, CPS, K)

    dstd = ei[1].reshape(NW, NSUP, CPSD, KD)
    degp = _deg_kernel(dstd)
    degt = degp.reshape(NC, N).T
    g1 = _mm1(x, W1, degt)
    p = _edge_kernel(g1, src, dst)
    g2 = _mm2(p, g1, degt, b1.reshape(1, D), W2)
    q = _edge_kernel(g2, src, dst)
    return _fin(q, g2, degt, b2.reshape(1, D))


# speedup vs baseline: 1.1430x; 1.0348x over previous
import functools

import jax
import jax.numpy as jnp
from jax import lax
from jax.experimental import pallas as pl
from jax.experimental.pallas import tpu as pltpu
from jax.experimental.pallas import tpu_sc as plsc

N = 10000
D = 128
E = 320000
NC = 2
NS = 16
NW = NC * NS
EPW = E // NW
K = 80
NCHUNK = EPW // K
NSUP = 5
CPS = NCHUNK // NSUP
NBUF = 4
PF = NBUF - 1
KD = 80
CPSD = EPW // KD // NSUP
RPT = N // NS
ZROWS = 125

_mesh = plsc.VectorSubcoreMesh(core_axis_name="c", subcore_axis_name="s")


@functools.partial(
    pl.kernel,
    out_type=jax.ShapeDtypeStruct((NC, 1, N), jnp.float32),
    mesh=_mesh,
    scratch_types=[
        pltpu.VMEM((CPSD, KD), jnp.int32),
        pltpu.VMEM((KD,), jnp.float32),
        pltpu.VMEM((N,), jnp.float32),
        pltpu.VMEM_SHARED((N,), jnp.float32),
    ],
)
def _deg_kernel(dst_hbm, out_hbm, didx, ones_v, zbuf, acc):
    c = lax.axis_index("c")
    s = lax.axis_index("s")
    wid = c * NS + s
    for i in range(KD // 16):
        ones_v[pl.ds(i * 16, 16)] = jnp.ones((16,), jnp.float32)

    @pl.when(s == 0)
    def _():
        def zr(i, carry):
            zbuf[pl.ds(i * 16, 16)] = jnp.zeros((16,), jnp.float32)
            return carry
        lax.fori_loop(0, N // 16, zr, 0)
        pltpu.sync_copy(zbuf, acc)

    plsc.subcore_barrier()

    def sup(sc, carry):
        pltpu.sync_copy(dst_hbm.at[wid, sc], didx)

        def body(j, carry2):
            pltpu.sync_copy(ones_v, acc.at[didx.at[j]], add=True)
            return carry2
        lax.fori_loop(0, CPSD, body, 0)
        return carry
    lax.fori_loop(0, NSUP, sup, 0)

    plsc.subcore_barrier()

    @pl.when(s == 0)
    def _():
        pltpu.sync_copy(acc, out_hbm.at[c, 0])


@functools.partial(
    pl.kernel,
    out_type=jax.ShapeDtypeStruct((NC, N, D), jnp.float32),
    mesh=_mesh,
    scratch_types=[
        pltpu.VMEM((CPS, K), jnp.int32),
        pltpu.VMEM((CPS, K), jnp.int32),
        pltpu.VMEM((NBUF, K, D), jnp.float32),
        pltpu.VMEM_SHARED((N, D), jnp.float32),
        pltpu.SemaphoreType.DMA((NBUF,)),
        pltpu.SemaphoreType.DMA((NBUF,)),
    ],
)
def _edge_kernel(g_hbm, src_hbm, dst_hbm, out_hbm, sidx, didx, rows, acc, gsem,
                 ssem):
    c = lax.axis_index("c")
    s = lax.axis_index("s")
    wid = c * NS + s

    def zr(i, carry):
        for jj in range(D // 16):
            rows[0, i, pl.ds(jj * 16, 16)] = jnp.zeros((16,), jnp.float32)
        return carry
    lax.fori_loop(0, K, zr, 0)
    for t in range(RPT // K):
        pltpu.sync_copy(rows.at[0], acc.at[pl.ds(s * RPT + t * K, K)])
    pltpu.sync_copy(rows.at[0, pl.ds(0, RPT % K)],
                    acc.at[pl.ds(s * RPT + (RPT // K) * K, RPT % K)])

    plsc.subcore_barrier()

    def sup(sc, carry):
        pltpu.sync_copy(src_hbm.at[wid, sc], sidx)
        pltpu.sync_copy(dst_hbm.at[wid, sc], didx)

        for i in range(PF):
            pltpu.async_copy(g_hbm.at[sidx.at[i]], rows.at[i], gsem.at[i])

        def body(j, carry2):
            rb = lax.rem(j, NBUF)
            nb = lax.rem(j + PF, NBUF)
            pltpu.make_async_copy(g_hbm.at[sidx.at[j]], rows.at[rb],
                                  gsem.at[rb]).wait()

            @pl.when(jnp.logical_and(j >= 1, j + PF < CPS))
            def _():
                pltpu.make_async_copy(rows.at[nb], acc.at[didx.at[j - 1]],
                                      ssem.at[nb]).wait()

            @pl.when(j + PF < CPS)
            def _():
                pltpu.async_copy(g_hbm.at[sidx.at[j + PF]], rows.at[nb],
                                 gsem.at[nb])

            pltpu.async_copy(rows.at[rb], acc.at[didx.at[j]], ssem.at[rb],
                             add=True)
            return carry2
        lax.fori_loop(0, CPS, body, 0)

        for jj in range(CPS - NBUF, CPS):
            pltpu.make_async_copy(rows.at[jj % NBUF], acc.at[didx.at[jj]],
                                  ssem.at[jj % NBUF]).wait()
        return carry
    lax.fori_loop(0, NSUP, sup, 0)

    plsc.subcore_barrier()
    WA = (N // NS) // 8 * 8

    @pl.when(s < NS - 1)
    def _():
        pltpu.sync_copy(acc.at[pl.ds(s * WA, WA)], out_hbm.at[c, pl.ds(s * WA, WA)])

    @pl.when(s == NS - 1)
    def _():
        pltpu.sync_copy(acc.at[pl.ds((NS - 1) * WA, N - (NS - 1) * WA)],
                        out_hbm.at[c, pl.ds((NS - 1) * WA, N - (NS - 1) * WA)])


BM = 2000


def _mm1_body(x_ref, w_ref, degt_ref, g_ref):
    deg = degt_ref[:, 0] + degt_ref[:, 1] + 1.0
    dis = lax.rsqrt(deg)
    h = jnp.dot(x_ref[...], w_ref[...], preferred_element_type=jnp.float32)
    g_ref[...] = h * dis[:, None]


def _mm2_body(p_ref, g1_ref, degt_ref, b1_ref, w_ref, g_ref):
    deg = degt_ref[:, 0] + degt_ref[:, 1] + 1.0
    dis = lax.rsqrt(deg)
    u = jnp.maximum(
        dis[:, None] * (p_ref[0] + p_ref[1] + g1_ref[...]) + b1_ref[...], 0.0)
    h = jnp.dot(u, w_ref[...], preferred_element_type=jnp.float32)
    g_ref[...] = h * dis[:, None]


def _fin_body(q_ref, g2_ref, degt_ref, b2_ref, z_ref):
    deg = degt_ref[:, 0] + degt_ref[:, 1] + 1.0
    dis = lax.rsqrt(deg)
    z_ref[...] = dis[:, None] * (q_ref[0] + q_ref[1] + g2_ref[...]) + b2_ref[...]


def _mm1(x, W1, degt):
    return pl.pallas_call(
        _mm1_body,
        grid=(N // BM,),
        in_specs=[
            pl.BlockSpec((BM, D), lambda i: (i, 0)),
            pl.BlockSpec((D, D), lambda i: (0, 0)),
            pl.BlockSpec((BM, 2), lambda i: (i, 0)),
        ],
        out_specs=pl.BlockSpec((BM, D), lambda i: (i, 0)),
        out_shape=jax.ShapeDtypeStruct((N, D), jnp.float32),
    )(x, W1, degt)


def _mm2(p, g1, degt, b1, W2):
    return pl.pallas_call(
        _mm2_body,
        grid=(N // BM,),
        in_specs=[
            pl.BlockSpec((NC, BM, D), lambda i: (0, i, 0)),
            pl.BlockSpec((BM, D), lambda i: (i, 0)),
            pl.BlockSpec((BM, 2), lambda i: (i, 0)),
            pl.BlockSpec((1, D), lambda i: (0, 0)),
            pl.BlockSpec((D, D), lambda i: (0, 0)),
        ],
        out_specs=pl.BlockSpec((BM, D), lambda i: (i, 0)),
        out_shape=jax.ShapeDtypeStruct((N, D), jnp.float32),
    )(p, g1, degt, b1, W2)


def _fin(q, g2, degt, b2):
    return pl.pallas_call(
        _fin_body,
        grid=(N // BM,),
        in_specs=[
            pl.BlockSpec((NC, BM, D), lambda i: (0, i, 0)),
            pl.BlockSpec((BM, D), lambda i: (i, 0)),
            pl.BlockSpec((BM, 2), lambda i: (i, 0)),
            pl.BlockSpec((1, D), lambda i: (0, 0)),
        ],
        out_specs=pl.BlockSpec((BM, D), lambda i: (i, 0)),
        out_shape=jax.ShapeDtypeStruct((N, D), jnp.float32),
    )(q, g2, degt, b2)


def kernel(x, edge_index, W1, b1, W2, b2):
    ei = edge_index.astype(jnp.int32)
    src = ei[0].reshape(NW, NSUP, CPS, K)
    dst = ei[1].reshape(NW, NSUP, CPS, K)

    dstd = ei[1].reshape(NW, NSUP, CPSD, KD)
    degp = _deg_kernel(dstd)
    degt = degp.reshape(NC, N).T
    g1 = _mm1(x, W1, degt)
    p = _edge_kernel(g1, src, dst)
    g2 = _mm2(p, g1, degt, b1.reshape(1, D), W2)
    q = _edge_kernel(g2, src, dst)
    return _fin(q, g2, degt, b2.reshape(1, D))


# speedup vs baseline: 1.1741x; 1.0272x over previous
import functools

import jax
import jax.numpy as jnp
from jax import lax
from jax.experimental import pallas as pl
from jax.experimental.pallas import tpu as pltpu
from jax.experimental.pallas import tpu_sc as plsc

N = 10000
D = 128
E = 320000
NC = 2
NS = 16
NW = NC * NS
EPW = E // NW
K = 80
NCHUNK = EPW // K
NSUP = 5
CPS = NCHUNK // NSUP
NBUF = 4
PF = NBUF - 1
KD = 80
CPSD = EPW // KD // NSUP
RPT = N // NS
ZROWS = 125

_mesh = plsc.VectorSubcoreMesh(core_axis_name="c", subcore_axis_name="s")


@functools.partial(
    pl.kernel,
    out_type=jax.ShapeDtypeStruct((NC, 1, N), jnp.float32),
    mesh=_mesh,
    scratch_types=[
        pltpu.VMEM((CPSD, KD), jnp.int32),
        pltpu.VMEM((KD,), jnp.float32),
        pltpu.VMEM((N,), jnp.float32),
        pltpu.VMEM_SHARED((N,), jnp.float32),
        pltpu.SemaphoreType.DMA,
    ],
)
def _deg_kernel(dst_hbm, out_hbm, didx, ones_v, zbuf, acc, dsem):
    c = lax.axis_index("c")
    s = lax.axis_index("s")
    wid = c * NS + s
    for i in range(KD // 16):
        ones_v[pl.ds(i * 16, 16)] = jnp.ones((16,), jnp.float32)

    @pl.when(s == 0)
    def _():
        def zr(i, carry):
            zbuf[pl.ds(i * 16, 16)] = jnp.zeros((16,), jnp.float32)
            return carry
        lax.fori_loop(0, N // 16, zr, 0)
        pltpu.sync_copy(zbuf, acc)

    plsc.subcore_barrier()

    def sup(sc, carry):
        pltpu.sync_copy(dst_hbm.at[wid, sc], didx)

        def body(j, carry2):
            pltpu.async_copy(ones_v, acc.at[didx.at[j]], dsem, add=True)
            return carry2
        lax.fori_loop(0, CPSD, body, 0)

        def drain(j, carry2):
            pltpu.make_async_copy(ones_v, acc.at[didx.at[0]], dsem).wait()
            return carry2
        lax.fori_loop(0, CPSD, drain, 0)
        return carry
    lax.fori_loop(0, NSUP, sup, 0)

    plsc.subcore_barrier()

    @pl.when(s == 0)
    def _():
        pltpu.sync_copy(acc, out_hbm.at[c, 0])


@functools.partial(
    pl.kernel,
    out_type=jax.ShapeDtypeStruct((NC, N, D), jnp.float32),
    mesh=_mesh,
    scratch_types=[
        pltpu.VMEM((CPS, K), jnp.int32),
        pltpu.VMEM((CPS, K), jnp.int32),
        pltpu.VMEM((NBUF, K, D), jnp.float32),
        pltpu.VMEM_SHARED((N, D), jnp.float32),
        pltpu.SemaphoreType.DMA((NBUF,)),
        pltpu.SemaphoreType.DMA((NBUF,)),
    ],
)
def _edge_kernel(g_hbm, src_hbm, dst_hbm, out_hbm, sidx, didx, rows, acc, gsem,
                 ssem):
    c = lax.axis_index("c")
    s = lax.axis_index("s")
    wid = c * NS + s

    def zr(i, carry):
        for jj in range(D // 16):
            rows[0, i, pl.ds(jj * 16, 16)] = jnp.zeros((16,), jnp.float32)
        return carry
    lax.fori_loop(0, K, zr, 0)
    for t in range(RPT // K):
        pltpu.sync_copy(rows.at[0], acc.at[pl.ds(s * RPT + t * K, K)])
    pltpu.sync_copy(rows.at[0, pl.ds(0, RPT % K)],
                    acc.at[pl.ds(s * RPT + (RPT // K) * K, RPT % K)])

    plsc.subcore_barrier()

    def sup(sc, carry):
        pltpu.sync_copy(src_hbm.at[wid, sc], sidx)
        pltpu.sync_copy(dst_hbm.at[wid, sc], didx)

        for i in range(PF):
            pltpu.async_copy(g_hbm.at[sidx.at[i]], rows.at[i], gsem.at[i])

        def body(j, carry2):
            rb = lax.rem(j, NBUF)
            nb = lax.rem(j + PF, NBUF)
            pltpu.make_async_copy(g_hbm.at[sidx.at[j]], rows.at[rb],
                                  gsem.at[rb]).wait()

            @pl.when(jnp.logical_and(j >= 1, j + PF < CPS))
            def _():
                pltpu.make_async_copy(rows.at[nb], acc.at[didx.at[j - 1]],
                                      ssem.at[nb]).wait()

            @pl.when(j + PF < CPS)
            def _():
                pltpu.async_copy(g_hbm.at[sidx.at[j + PF]], rows.at[nb],
                                 gsem.at[nb])

            pltpu.async_copy(rows.at[rb], acc.at[didx.at[j]], ssem.at[rb],
                             add=True)
            return carry2
        lax.fori_loop(0, CPS, body, 0)

        for jj in range(CPS - NBUF, CPS):
            pltpu.make_async_copy(rows.at[jj % NBUF], acc.at[didx.at[jj]],
                                  ssem.at[jj % NBUF]).wait()
        return carry
    lax.fori_loop(0, NSUP, sup, 0)

    plsc.subcore_barrier()
    WA = (N // NS) // 8 * 8

    @pl.when(s < NS - 1)
    def _():
        pltpu.sync_copy(acc.at[pl.ds(s * WA, WA)], out_hbm.at[c, pl.ds(s * WA, WA)])

    @pl.when(s == NS - 1)
    def _():
        pltpu.sync_copy(acc.at[pl.ds((NS - 1) * WA, N - (NS - 1) * WA)],
                        out_hbm.at[c, pl.ds((NS - 1) * WA, N - (NS - 1) * WA)])


BM = 2000


def _mm1_body(x_ref, w_ref, degt_ref, g_ref):
    deg = degt_ref[:, 0] + degt_ref[:, 1] + 1.0
    dis = lax.rsqrt(deg)
    h = jnp.dot(x_ref[...], w_ref[...], preferred_element_type=jnp.float32)
    g_ref[...] = h * dis[:, None]


def _mm2_body(p_ref, g1_ref, degt_ref, b1_ref, w_ref, g_ref):
    deg = degt_ref[:, 0] + degt_ref[:, 1] + 1.0
    dis = lax.rsqrt(deg)
    u = jnp.maximum(
        dis[:, None] * (p_ref[0] + p_ref[1] + g1_ref[...]) + b1_ref[...], 0.0)
    h = jnp.dot(u, w_ref[...], preferred_element_type=jnp.float32)
    g_ref[...] = h * dis[:, None]


def _fin_body(q_ref, g2_ref, degt_ref, b2_ref, z_ref):
    deg = degt_ref[:, 0] + degt_ref[:, 1] + 1.0
    dis = lax.rsqrt(deg)
    z_ref[...] = dis[:, None] * (q_ref[0] + q_ref[1] + g2_ref[...]) + b2_ref[...]


def _mm1(x, W1, degt):
    return pl.pallas_call(
        _mm1_body,
        grid=(N // BM,),
        in_specs=[
            pl.BlockSpec((BM, D), lambda i: (i, 0)),
            pl.BlockSpec((D, D), lambda i: (0, 0)),
            pl.BlockSpec((BM, 2), lambda i: (i, 0)),
        ],
        out_specs=pl.BlockSpec((BM, D), lambda i: (i, 0)),
        out_shape=jax.ShapeDtypeStruct((N, D), jnp.float32),
    )(x, W1, degt)


def _mm2(p, g1, degt, b1, W2):
    return pl.pallas_call(
        _mm2_body,
        grid=(N // BM,),
        in_specs=[
            pl.BlockSpec((NC, BM, D), lambda i: (0, i, 0)),
            pl.BlockSpec((BM, D), lambda i: (i, 0)),
            pl.BlockSpec((BM, 2), lambda i: (i, 0)),
            pl.BlockSpec((1, D), lambda i: (0, 0)),
            pl.BlockSpec((D, D), lambda i: (0, 0)),
        ],
        out_specs=pl.BlockSpec((BM, D), lambda i: (i, 0)),
        out_shape=jax.ShapeDtypeStruct((N, D), jnp.float32),
    )(p, g1, degt, b1, W2)


def _fin(q, g2, degt, b2):
    return pl.pallas_call(
        _fin_body,
        grid=(N // BM,),
        in_specs=[
            pl.BlockSpec((NC, BM, D), lambda i: (0, i, 0)),
            pl.BlockSpec((BM, D), lambda i: (i, 0)),
            pl.BlockSpec((BM, 2), lambda i: (i, 0)),
            pl.BlockSpec((1, D), lambda i: (0, 0)),
        ],
        out_specs=pl.BlockSpec((BM, D), lambda i: (i, 0)),
        out_shape=jax.ShapeDtypeStruct((N, D), jnp.float32),
    )(q, g2, degt, b2)


def kernel(x, edge_index, W1, b1, W2, b2):
    ei = edge_index.astype(jnp.int32)
    src = ei[0].reshape(NW, NSUP, CPS, K)
    dst = ei[1].reshape(NW, NSUP, CPS, K)

    dstd = ei[1].reshape(NW, NSUP, CPSD, KD)
    degp = _deg_kernel(dstd)
    degt = degp.reshape(NC, N).T
    g1 = _mm1(x, W1, degt)
    p = _edge_kernel(g1, src, dst)
    g2 = _mm2(p, g1, degt, b1.reshape(1, D), W2)
    q = _edge_kernel(g2, src, dst)
    return _fin(q, g2, degt, b2.reshape(1, D))
